# Initial kernel scaffold; baseline (speedup 1.0000x reference)
import jax
import jax.numpy as jnp
from jax.experimental import pallas as pl


def kernel(x, edge_index, edge_values, W, b, gamma, beta):
    raise NotImplementedError("write your pallas kernel here")


# speedup vs baseline: 3.8697x; 3.8697x over previous
import functools

import jax
import jax.numpy as jnp
from jax import lax
from jax.experimental import pallas as pl
from jax.experimental.pallas import tpu as pltpu
from jax.experimental.pallas import tpu_sc as plsc

N = 10000
D = 128
E = 320000

NC = 2
NS = 16
NW = NC * NS

CHUNK = 128
CHUNKS_PER_W = 79
EPW = CHUNK * CHUNKS_PER_W
EPAD = EPW * NW

ROWS_PER_TILE = N // NS


def _sc_body(x_hbm, cols_hbm, dst_hbm, vals_hbm, part_hbm,
             colsv, dstv, valsv, rowsv, acc, sem):
    c = lax.axis_index("c")
    s = lax.axis_index("s")
    wid = c * NS + s
    ebase = wid * EPW

    @pl.loop(0, CHUNK)
    def _zero(r):
        for d in range(D // 16):
            rowsv[r, pl.ds(d * 16, 16)] = jnp.zeros((16,), jnp.float32)

    for j in range(4):
        pltpu.sync_copy(rowsv.at[:],
                        acc.at[pl.ds(s * ROWS_PER_TILE + j * CHUNK, CHUNK)])
    pltpu.sync_copy(
        rowsv.at[pl.ds(0, ROWS_PER_TILE - 4 * CHUNK)],
        acc.at[pl.ds(s * ROWS_PER_TILE + 4 * CHUNK, ROWS_PER_TILE - 4 * CHUNK)])

    plsc.subcore_barrier()

    @pl.loop(0, CHUNKS_PER_W)
    def _chunk(i):
        off = ebase + i * CHUNK
        pltpu.sync_copy(cols_hbm.at[pl.ds(off, CHUNK)], colsv)
        pltpu.sync_copy(vals_hbm.at[pl.ds(off, CHUNK)], valsv)
        pltpu.sync_copy(dst_hbm.at[pl.ds(off, CHUNK)], dstv)
        pltpu.async_copy(x_hbm.at[colsv], rowsv, sem).wait()

        @pl.loop(0, CHUNK // 16)
        def _scale(g):
            vv = valsv[pl.ds(g * 16, 16)]
            for j in range(16):
                vb = vv[j]
                e = g * 16 + j
                for d in range(D // 16):
                    sl = pl.ds(d * 16, 16)
                    rowsv[e, sl] = rowsv[e, sl] * vb

        pltpu.sync_copy(rowsv, acc.at[dstv], add=True)

    plsc.subcore_barrier()

    WR = 624
    pltpu.sync_copy(acc.at[pl.ds(s * WR, WR)],
                    part_hbm.at[c, pl.ds(s * WR, WR)])

    @pl.when(s == 0)
    def _tail():
        pltpu.sync_copy(acc.at[pl.ds(NS * WR, N - NS * WR)],
                        part_hbm.at[c, pl.ds(NS * WR, N - NS * WR)])


def _sc_neighbor_sum(x, cols, dst, vals):
    mesh = plsc.VectorSubcoreMesh(core_axis_name="c", subcore_axis_name="s",
                                  num_cores=NC, num_subcores=NS)

    fn = pl.kernel(
        _sc_body,
        out_type=jax.ShapeDtypeStruct((NC, N, D), jnp.float32),
        mesh=mesh,
        scratch_types=[
            pltpu.VMEM((CHUNK,), jnp.int32),
            pltpu.VMEM((CHUNK,), jnp.int32),
            pltpu.VMEM((CHUNK,), jnp.float32),
            pltpu.VMEM((CHUNK, D), jnp.float32),
            pltpu.VMEM_SHARED((N, D), jnp.float32),
            pltpu.SemaphoreType.DMA,
        ],
    )
    return fn(x, cols, dst, vals)


def _tc_body(x_ref, p0_ref, p1_ref, w1_ref, w2_ref, b_ref, g_ref, be_ref,
             o_ref):
    xb = x_ref[...]
    xn = p0_ref[...] + p1_ref[...]
    h = (jnp.dot(xb, w1_ref[...], preferred_element_type=jnp.float32)
         + jnp.dot(xn, w2_ref[...], preferred_element_type=jnp.float32)
         + b_ref[...])
    y = jnp.maximum(h, 0.0) + xb
    mean = jnp.mean(y, axis=1, keepdims=True)
    yc = y - mean
    var = jnp.mean(yc * yc, axis=1, keepdims=True)
    ynorm = yc * lax.rsqrt(var + 1e-5)
    o_ref[...] = ynorm * g_ref[...] + be_ref[...]


def _tc_combine(x, p0, p1, w1t, w2t, b, gamma, beta):
    BLK = 2000
    grid = (N // BLK,)
    row_spec = pl.BlockSpec((BLK, D), lambda i: (i, 0))
    full_spec = pl.BlockSpec((D, D), lambda i: (0, 0))
    vec_spec = pl.BlockSpec((1, D), lambda i: (0, 0))
    return pl.pallas_call(
        _tc_body,
        grid=grid,
        in_specs=[row_spec, row_spec, row_spec, full_spec, full_spec,
                  vec_spec, vec_spec, vec_spec],
        out_specs=row_spec,
        out_shape=jax.ShapeDtypeStruct((N, D), jnp.float32),
    )(x, p0, p1, w1t, w2t, b.reshape(1, D), gamma.reshape(1, D),
      beta.reshape(1, D))


@jax.jit
def kernel(x, edge_index, edge_values, W, b, gamma, beta):
    dst = edge_index[0]
    cols = edge_index[1]
    pad = EPAD - E
    cols_p = jnp.pad(cols, (0, pad))
    dst_p = jnp.pad(dst, (0, pad))
    vals_p = jnp.pad(edge_values, (0, pad))

    partials = _sc_neighbor_sum(x, cols_p, dst_p, vals_p)

    wt = W.T
    return _tc_combine(x, partials[0], partials[1], wt[:D], wt[D:],
                       b, gamma, beta)
